# baseline (speedup 1.0000x reference)
import jax
import jax.numpy as jnp
from jax.experimental import pallas as pl
from jax.experimental.pallas import tpu as pltpu

_CELLS = 2


def _encoder_body(x_ref, bn_scale_ref, bn_shift_ref, w0_ref, b0_ref,
                  w1_ref, b1_ref, o_ref, buf_ref):
    H = x_ref.shape[1]
    W = x_ref.shape[2]
    C = x_ref.shape[3]
    HW = H * W

    buf_ref[...] = jnp.zeros(buf_ref.shape, buf_ref.dtype)

    def bn_conv(x2d, bn_row, c, w_ref, b):
        scale = bn_scale_ref[bn_row:bn_row + 1, :]
        shift = bn_shift_ref[bn_row:bn_row + 1, :]
        bnx = (x2d * scale + shift).astype(jnp.bfloat16).reshape(H, W, C)
        buf_ref[1:H + 1, :, C:2 * C] = bnx
        buf_ref[1:H + 1, 1:W, 0:C] = bnx[:, :W - 1, :]
        buf_ref[1:H + 1, 0:W - 1, 2 * C:3 * C] = bnx[:, 1:, :]
        acc = b
        for kh in range(3):
            tap3 = buf_ref[kh:kh + H, :, :].reshape(HW, 3 * C)
            wk = w_ref[c, kh * 3 * C:(kh + 1) * 3 * C, :]
            acc = acc + jnp.dot(tap3, wk, preferred_element_type=jnp.float32)
        return acc

    cell_in = x_ref[0].reshape(HW, C).astype(jnp.float32)
    for c in range(_CELLS):
        y0 = bn_conv(cell_in, 2 * c + 0, c, w0_ref, b0_ref[c])
        node1 = jnp.maximum(y0[:, :C], 0.0)
        y1 = bn_conv(node1, 2 * c + 1, c, w1_ref, b1_ref[c])
        cell_in = y0[:, C:] + jnp.maximum(y1, 0.0)

    o_ref[0] = jnp.sum(cell_in, axis=0, keepdims=True) * (1.0 / HW)


def _head_body(p_ref, hw_ref, hb_ref, o_ref):
    o_ref[...] = jnp.dot(p_ref[...], hw_ref[...],
                         preferred_element_type=jnp.float32) + hb_ref[...]


def kernel(x, bn_scale, bn_shift, w0, b0, w1, b1, head_w, head_b):
    x = jnp.transpose(x, (0, 2, 3, 1)).astype(jnp.bfloat16)
    B, H, W, C = x.shape
    K = head_w.shape[1]
    nine_c = 9 * C

    pooled = pl.pallas_call(
        _encoder_body,
        out_shape=jax.ShapeDtypeStruct((B, 1, C), jnp.float32),
        grid=(B,),
        in_specs=[
            pl.BlockSpec((1, H, W, C), lambda b: (b, 0, 0, 0)),
            pl.BlockSpec((2 * _CELLS, C), lambda b: (0, 0)),
            pl.BlockSpec((2 * _CELLS, C), lambda b: (0, 0)),
            pl.BlockSpec((_CELLS, nine_c, 2 * C), lambda b: (0, 0, 0)),
            pl.BlockSpec((_CELLS, 1, 2 * C), lambda b: (0, 0, 0)),
            pl.BlockSpec((_CELLS, nine_c, C), lambda b: (0, 0, 0)),
            pl.BlockSpec((_CELLS, 1, C), lambda b: (0, 0, 0)),
        ],
        out_specs=pl.BlockSpec((1, 1, C), lambda b: (b, 0, 0)),
        scratch_shapes=[
            pltpu.VMEM((H + 2, W, 3 * C), jnp.bfloat16),
        ],
        compiler_params=pltpu.CompilerParams(dimension_semantics=("parallel",)),
    )(x, bn_scale, bn_shift, w0.astype(jnp.bfloat16), b0,
      w1.astype(jnp.bfloat16), b1)

    logits = pl.pallas_call(
        _head_body,
        out_shape=jax.ShapeDtypeStruct((B, K), jnp.float32),
    )(pooled.reshape(B, C), head_w, head_b)
    return logits


# speedup vs baseline: 1.0697x; 1.0697x over previous
import jax
import jax.numpy as jnp
from jax.experimental import pallas as pl
from jax.experimental.pallas import tpu as pltpu

_CELLS = 2


def _encoder_body(x_ref, bn_scale_ref, bn_shift_ref, w0_ref, b0_ref,
                  w1_ref, b1_ref, o_ref, stg_a, stg_b, pat_a, pat_b):
    H = x_ref.shape[1]
    W = x_ref.shape[2]
    C = x_ref.shape[3]
    HW = H * W

    stg_a[...] = jnp.zeros(stg_a.shape, stg_a.dtype)
    stg_b[...] = jnp.zeros(stg_b.shape, stg_b.dtype)

    def bn_conv(stg, pat, x2d, bn_row, w, b):
        scale = bn_scale_ref[bn_row:bn_row + 1, :]
        shift = bn_shift_ref[bn_row:bn_row + 1, :]
        bnx = (x2d * scale + shift).astype(jnp.bfloat16).reshape(H, W, C)
        stg[1:H + 1, :, C:2 * C] = bnx
        stg[1:H + 1, 1:W, 0:C] = bnx[:, :W - 1, :]
        stg[1:H + 1, 0:W - 1, 2 * C:3 * C] = bnx[:, 1:, :]
        pat[:, :, 0:3 * C] = stg[0:H, :, :]
        pat[:, :, 3 * C:6 * C] = stg[1:H + 1, :, :]
        pat[:, :, 6 * C:9 * C] = stg[2:H + 2, :, :]
        return jnp.dot(pat[...].reshape(HW, 9 * C), w,
                       preferred_element_type=jnp.float32) + b

    for i, (stg, pat) in enumerate(((stg_a, pat_a), (stg_b, pat_b))):
        cell_in = x_ref[i].reshape(HW, C).astype(jnp.float32)
        for c in range(_CELLS):
            y0 = bn_conv(stg, pat, cell_in, 2 * c + 0, w0_ref[c], b0_ref[c])
            node1 = jnp.maximum(y0[:, :C], 0.0)
            y1 = bn_conv(stg, pat, node1, 2 * c + 1, w1_ref[c], b1_ref[c])
            cell_in = y0[:, C:] + jnp.maximum(y1, 0.0)
        o_ref[i] = jnp.sum(cell_in, axis=0, keepdims=True) * (1.0 / HW)


def _head_body(p_ref, hw_ref, hb_ref, o_ref):
    o_ref[...] = jnp.dot(p_ref[...], hw_ref[...],
                         preferred_element_type=jnp.float32) + hb_ref[...]


def kernel(x, bn_scale, bn_shift, w0, b0, w1, b1, head_w, head_b):
    x = jnp.transpose(x, (0, 2, 3, 1)).astype(jnp.bfloat16)
    B, H, W, C = x.shape
    K = head_w.shape[1]
    nine_c = 9 * C

    pooled = pl.pallas_call(
        _encoder_body,
        out_shape=jax.ShapeDtypeStruct((B, 1, C), jnp.float32),
        grid=(B // 2,),
        in_specs=[
            pl.BlockSpec((2, H, W, C), lambda b: (b, 0, 0, 0)),
            pl.BlockSpec((2 * _CELLS, C), lambda b: (0, 0)),
            pl.BlockSpec((2 * _CELLS, C), lambda b: (0, 0)),
            pl.BlockSpec((_CELLS, nine_c, 2 * C), lambda b: (0, 0, 0)),
            pl.BlockSpec((_CELLS, 1, 2 * C), lambda b: (0, 0, 0)),
            pl.BlockSpec((_CELLS, nine_c, C), lambda b: (0, 0, 0)),
            pl.BlockSpec((_CELLS, 1, C), lambda b: (0, 0, 0)),
        ],
        out_specs=pl.BlockSpec((2, 1, C), lambda b: (b, 0, 0)),
        scratch_shapes=[
            pltpu.VMEM((H + 2, W, 3 * C), jnp.bfloat16),
            pltpu.VMEM((H + 2, W, 3 * C), jnp.bfloat16),
            pltpu.VMEM((H, W, 9 * C), jnp.bfloat16),
            pltpu.VMEM((H, W, 9 * C), jnp.bfloat16),
        ],
        compiler_params=pltpu.CompilerParams(dimension_semantics=("parallel",)),
    )(x, bn_scale, bn_shift, w0.astype(jnp.bfloat16), b0,
      w1.astype(jnp.bfloat16), b1)

    logits = pl.pallas_call(
        _head_body,
        out_shape=jax.ShapeDtypeStruct((B, K), jnp.float32),
    )(pooled.reshape(B, C), head_w, head_b)
    return logits


# speedup vs baseline: 1.5922x; 1.4885x over previous
import jax
import jax.numpy as jnp
from jax.experimental import pallas as pl
from jax.experimental.pallas import tpu as pltpu

_CELLS = 2


def _encoder_body(x_ref, bn_scale_ref, bn_shift_ref, w0_ref, b0_ref,
                  w1_ref, b1_ref, o_ref, stg_a, stg_b, pat_a, pat_b):
    H = x_ref.shape[1]
    W = x_ref.shape[2]
    C = x_ref.shape[3]
    HW = H * W

    stg_a[...] = jnp.zeros(stg_a.shape, stg_a.dtype)
    stg_b[...] = jnp.zeros(stg_b.shape, stg_b.dtype)

    def bn_conv(stg, pat, x2d, bn_row, w, b):
        scale = bn_scale_ref[bn_row:bn_row + 1, :]
        shift = bn_shift_ref[bn_row:bn_row + 1, :]
        bnx = (x2d * scale + shift).astype(jnp.bfloat16).reshape(H, W, C)
        stg[1:H + 1, :, C:2 * C] = bnx
        stg[1:H + 1, 1:W, 0:C] = bnx[:, :W - 1, :]
        stg[1:H + 1, 0:W - 1, 2 * C:3 * C] = bnx[:, 1:, :]
        pat[:, :, 0:3 * C] = stg[0:H, :, :]
        pat[:, :, 3 * C:6 * C] = stg[1:H + 1, :, :]
        pat[:, :, 6 * C:9 * C] = stg[2:H + 2, :, :]
        return jnp.dot(pat[...].reshape(HW, 9 * C), w,
                       preferred_element_type=jnp.float32) + b

    cell_a = x_ref[0].reshape(HW, C).astype(jnp.float32)
    cell_b = x_ref[1].reshape(HW, C).astype(jnp.float32)
    for c in range(_CELLS):
        y0a = bn_conv(stg_a, pat_a, cell_a, 2 * c + 0, w0_ref[c], b0_ref[c])
        y0b = bn_conv(stg_b, pat_b, cell_b, 2 * c + 0, w0_ref[c], b0_ref[c])
        n1a = jnp.maximum(y0a[:, :C], 0.0)
        n1b = jnp.maximum(y0b[:, :C], 0.0)
        y1a = bn_conv(stg_a, pat_a, n1a, 2 * c + 1, w1_ref[c], b1_ref[c])
        y1b = bn_conv(stg_b, pat_b, n1b, 2 * c + 1, w1_ref[c], b1_ref[c])
        cell_a = y0a[:, C:] + jnp.maximum(y1a, 0.0)
        cell_b = y0b[:, C:] + jnp.maximum(y1b, 0.0)
    o_ref[0] = jnp.sum(cell_a, axis=0, keepdims=True) * (1.0 / HW)
    o_ref[1] = jnp.sum(cell_b, axis=0, keepdims=True) * (1.0 / HW)


def _head_body(p_ref, hw_ref, hb_ref, o_ref):
    o_ref[...] = jnp.dot(p_ref[...], hw_ref[...],
                         preferred_element_type=jnp.float32) + hb_ref[...]


def kernel(x, bn_scale, bn_shift, w0, b0, w1, b1, head_w, head_b):
    x = jnp.transpose(x, (0, 2, 3, 1)).astype(jnp.bfloat16)
    B, H, W, C = x.shape
    K = head_w.shape[1]
    nine_c = 9 * C

    pooled = pl.pallas_call(
        _encoder_body,
        out_shape=jax.ShapeDtypeStruct((B, 1, C), jnp.float32),
        grid=(B // 2,),
        in_specs=[
            pl.BlockSpec((2, H, W, C), lambda b: (b, 0, 0, 0)),
            pl.BlockSpec((2 * _CELLS, C), lambda b: (0, 0)),
            pl.BlockSpec((2 * _CELLS, C), lambda b: (0, 0)),
            pl.BlockSpec((_CELLS, nine_c, 2 * C), lambda b: (0, 0, 0)),
            pl.BlockSpec((_CELLS, 1, 2 * C), lambda b: (0, 0, 0)),
            pl.BlockSpec((_CELLS, nine_c, C), lambda b: (0, 0, 0)),
            pl.BlockSpec((_CELLS, 1, C), lambda b: (0, 0, 0)),
        ],
        out_specs=pl.BlockSpec((2, 1, C), lambda b: (b, 0, 0)),
        scratch_shapes=[
            pltpu.VMEM((H + 2, W, 3 * C), jnp.bfloat16),
            pltpu.VMEM((H + 2, W, 3 * C), jnp.bfloat16),
            pltpu.VMEM((H, W, 9 * C), jnp.bfloat16),
            pltpu.VMEM((H, W, 9 * C), jnp.bfloat16),
        ],
        compiler_params=pltpu.CompilerParams(dimension_semantics=("parallel",)),
    )(x, bn_scale, bn_shift, w0.astype(jnp.bfloat16), b0,
      w1.astype(jnp.bfloat16), b1)

    logits = pl.pallas_call(
        _head_body,
        out_shape=jax.ShapeDtypeStruct((B, K), jnp.float32),
    )(pooled.reshape(B, C), head_w, head_b)
    return logits


# speedup vs baseline: 1.6198x; 1.0173x over previous
import jax
import jax.numpy as jnp
from jax.experimental import pallas as pl
from jax.experimental.pallas import tpu as pltpu

_CELLS = 2
_IPS = 4


def _encoder_body(x_ref, bn_scale_ref, bn_shift_ref, w0_ref, b0_ref,
                  w1_ref, b1_ref, o_ref, *scratch):
    H = x_ref.shape[1]
    W = x_ref.shape[2]
    C = x_ref.shape[3]
    HW = H * W
    stgs = scratch[:_IPS]
    pats = scratch[_IPS:]

    for stg in stgs:
        stg[...] = jnp.zeros(stg.shape, stg.dtype)

    def bn_conv(stg, pat, x2d, bn_row, w, b):
        scale = bn_scale_ref[bn_row:bn_row + 1, :]
        shift = bn_shift_ref[bn_row:bn_row + 1, :]
        bnx = (x2d * scale + shift).astype(jnp.bfloat16).reshape(H, W, C)
        stg[1:H + 1, :, C:2 * C] = bnx
        stg[1:H + 1, 1:W, 0:C] = bnx[:, :W - 1, :]
        stg[1:H + 1, 0:W - 1, 2 * C:3 * C] = bnx[:, 1:, :]
        pat[:, :, 0:3 * C] = stg[0:H, :, :]
        pat[:, :, 3 * C:6 * C] = stg[1:H + 1, :, :]
        pat[:, :, 6 * C:9 * C] = stg[2:H + 2, :, :]
        return jnp.dot(pat[...].reshape(HW, 9 * C), w,
                       preferred_element_type=jnp.float32) + b

    cells = [x_ref[i].reshape(HW, C).astype(jnp.float32) for i in range(_IPS)]
    for c in range(_CELLS):
        y0 = [bn_conv(stgs[i], pats[i], cells[i], 2 * c + 0,
                      w0_ref[c], b0_ref[c]) for i in range(_IPS)]
        n1 = [jnp.maximum(y0[i][:, :C], 0.0) for i in range(_IPS)]
        y1 = [bn_conv(stgs[i], pats[i], n1[i], 2 * c + 1,
                      w1_ref[c], b1_ref[c]) for i in range(_IPS)]
        cells = [y0[i][:, C:] + jnp.maximum(y1[i], 0.0) for i in range(_IPS)]
    for i in range(_IPS):
        o_ref[i] = jnp.sum(cells[i], axis=0, keepdims=True) * (1.0 / HW)


def _head_body(p_ref, hw_ref, hb_ref, o_ref):
    o_ref[...] = jnp.dot(p_ref[...], hw_ref[...],
                         preferred_element_type=jnp.float32) + hb_ref[...]


def kernel(x, bn_scale, bn_shift, w0, b0, w1, b1, head_w, head_b):
    x = jnp.transpose(x, (0, 2, 3, 1)).astype(jnp.bfloat16)
    B, H, W, C = x.shape
    K = head_w.shape[1]
    nine_c = 9 * C

    pooled = pl.pallas_call(
        _encoder_body,
        out_shape=jax.ShapeDtypeStruct((B, 1, C), jnp.float32),
        grid=(B // _IPS,),
        in_specs=[
            pl.BlockSpec((_IPS, H, W, C), lambda b: (b, 0, 0, 0)),
            pl.BlockSpec((2 * _CELLS, C), lambda b: (0, 0)),
            pl.BlockSpec((2 * _CELLS, C), lambda b: (0, 0)),
            pl.BlockSpec((_CELLS, nine_c, 2 * C), lambda b: (0, 0, 0)),
            pl.BlockSpec((_CELLS, 1, 2 * C), lambda b: (0, 0, 0)),
            pl.BlockSpec((_CELLS, nine_c, C), lambda b: (0, 0, 0)),
            pl.BlockSpec((_CELLS, 1, C), lambda b: (0, 0, 0)),
        ],
        out_specs=pl.BlockSpec((_IPS, 1, C), lambda b: (b, 0, 0)),
        scratch_shapes=(
            [pltpu.VMEM((H + 2, W, 3 * C), jnp.bfloat16) for _ in range(_IPS)]
            + [pltpu.VMEM((H, W, 9 * C), jnp.bfloat16) for _ in range(_IPS)]),
        compiler_params=pltpu.CompilerParams(dimension_semantics=("parallel",)),
    )(x, bn_scale, bn_shift, w0.astype(jnp.bfloat16), b0,
      w1.astype(jnp.bfloat16), b1)

    logits = pl.pallas_call(
        _head_body,
        out_shape=jax.ShapeDtypeStruct((B, K), jnp.float32),
    )(pooled.reshape(B, C), head_w, head_b)
    return logits
